# baseline (speedup 1.0000x reference)
import functools

import jax
import jax.numpy as jnp
from jax.experimental import pallas as pl
from jax.experimental.pallas import tpu as pltpu

_LANE = 128
_SUBLANE = 8


def _mse_partial_kernel(x_ref, y_ref, w1_ref, b1_ref, w2_ref, b2_ref,
                        part_ref, *, tile_b):
    y = y_ref[...]
    w2 = w2_ref[...]
    b2 = b2_ref[...]

    ht = jax.lax.dot_general(
        w1_ref[...], x_ref[...], (((0,), (1,)), ((), ())),
        preferred_element_type=jnp.float32)
    ht = jax.nn.sigmoid(ht + b1_ref[...].T)

    s = jax.lax.dot_general(ht, ht, (((1,), (1,)), ((), ())),
                            preferred_element_type=jnp.float32)
    w2g = jax.lax.dot_general(w2, w2, (((1,), (1,)), ((), ())),
                              preferred_element_type=jnp.float32)
    sum_a2 = jnp.sum(s * w2g)

    g = jnp.dot(ht, y, preferred_element_type=jnp.float32)
    cross_ay = jnp.sum(g * w2)

    hsum = jnp.sum(ht, axis=1, keepdims=True)
    w2b2 = jnp.dot(w2, b2.T, preferred_element_type=jnp.float32)
    cross_ab2 = jnp.sum(hsum * w2b2)

    b2sq = jnp.sum(b2 * b2)
    y_term = jnp.sum(y * (y - 2.0 * b2))

    part = (sum_a2 + 2.0 * cross_ab2 + tile_b * b2sq
            - 2.0 * cross_ay + y_term)

    sub = jax.lax.broadcasted_iota(jnp.int32, part_ref.shape, 1)
    lane = jax.lax.broadcasted_iota(jnp.int32, part_ref.shape, 2)
    part_ref[...] = jnp.where((sub == 0) & (lane == 0), part, 0.0)


def _pick_tile(b):
    for tb in (16384, 8192, 4096, 2048, 1024, 512, 256, 128, 64, 32, 16, 8):
        if b % tb == 0:
            return tb
    return b


def kernel(x, y, w1, b1, w2, b2):
    B, D = x.shape
    H = w1.shape[1]
    O = w2.shape[1]

    TB = _pick_tile(B)
    nb = B // TB

    def resident(shape):
        return pl.BlockSpec(shape, lambda i: (0, 0))

    block_bytes = (TB * D + TB * O) * 4
    vmem_limit = min(64 * 1024 * 1024, 2 * block_bytes + (8 << 20))

    partials = pl.pallas_call(
        functools.partial(_mse_partial_kernel, tile_b=float(TB)),
        out_shape=jax.ShapeDtypeStruct((nb, _SUBLANE, _LANE), jnp.float32),
        grid=(nb,),
        in_specs=[
            pl.BlockSpec((TB, D), lambda i: (i, 0)),
            pl.BlockSpec((TB, O), lambda i: (i, 0)),
            resident((D, H)),
            resident((1, H)),
            resident((H, O)),
            resident((1, O)),
        ],
        out_specs=pl.BlockSpec((1, _SUBLANE, _LANE), lambda i: (i, 0, 0)),
        compiler_params=pltpu.CompilerParams(
            dimension_semantics=("parallel",),
            vmem_limit_bytes=vmem_limit,
        ),
    )(x, y, w1, b1, w2, b2)
    return jnp.sum(partials) / jnp.float32(B * O)


# speedup vs baseline: 1.0253x; 1.0253x over previous
import functools

import jax
import jax.numpy as jnp
from jax.experimental import pallas as pl
from jax.experimental.pallas import tpu as pltpu

_LANE = 128
_SUBLANE = 8


def _mse_partial_kernel(x_ref, y_ref, w1t_ref, b1_ref, w2_ref, b2_ref,
                        part_ref, *, tile_b):
    y = y_ref[...]
    w2 = w2_ref[...]
    b2 = b2_ref[...]

    ht = jax.lax.dot_general(
        w1t_ref[...], x_ref[...], (((1,), (1,)), ((), ())),
        preferred_element_type=jnp.float32)
    ht = jax.nn.sigmoid(ht + b1_ref[...].T)

    s = jax.lax.dot_general(ht, ht, (((1,), (1,)), ((), ())),
                            preferred_element_type=jnp.float32)
    w2g = jax.lax.dot_general(w2, w2, (((1,), (1,)), ((), ())),
                              preferred_element_type=jnp.float32)
    sum_a2 = jnp.sum(s * w2g)

    g = jnp.dot(ht, y, preferred_element_type=jnp.float32)
    cross_ay = jnp.sum(g * w2)

    hsum = jnp.sum(ht, axis=1, keepdims=True)
    w2b2 = jnp.dot(w2, b2.T, preferred_element_type=jnp.float32)
    cross_ab2 = jnp.sum(hsum * w2b2)

    b2sq = jnp.sum(b2 * b2)
    y_term = jnp.sum(y * (y - 2.0 * b2))

    part = (sum_a2 + 2.0 * cross_ab2 + tile_b * b2sq
            - 2.0 * cross_ay + y_term)

    sub = jax.lax.broadcasted_iota(jnp.int32, part_ref.shape, 1)
    lane = jax.lax.broadcasted_iota(jnp.int32, part_ref.shape, 2)
    part_ref[...] = jnp.where((sub == 0) & (lane == 0), part, 0.0)


def _pick_tile(b):
    for tb in (16384, 8192, 4096, 2048, 1024, 512, 256, 128, 64, 32, 16, 8):
        if b % tb == 0:
            return tb
    return b


def kernel(x, y, w1, b1, w2, b2):
    B, D = x.shape
    H = w1.shape[1]
    O = w2.shape[1]

    TB = _pick_tile(B)
    nb = B // TB

    def resident(shape):
        return pl.BlockSpec(shape, lambda i: (0, 0))

    block_bytes = (TB * D + TB * O) * 4
    vmem_limit = min(64 * 1024 * 1024, 2 * block_bytes + (8 << 20))

    partials = pl.pallas_call(
        functools.partial(_mse_partial_kernel, tile_b=float(TB)),
        out_shape=jax.ShapeDtypeStruct((nb, _SUBLANE, _LANE), jnp.float32),
        grid=(nb,),
        in_specs=[
            pl.BlockSpec((TB, D), lambda i: (i, 0)),
            pl.BlockSpec((TB, O), lambda i: (i, 0)),
            resident((H, D)),
            resident((1, H)),
            resident((H, O)),
            resident((1, O)),
        ],
        out_specs=pl.BlockSpec((1, _SUBLANE, _LANE), lambda i: (i, 0, 0)),
        compiler_params=pltpu.CompilerParams(
            dimension_semantics=("parallel",),
            vmem_limit_bytes=vmem_limit,
        ),
    )(x, y, w1.T, b1, w2, b2)
    return jnp.sum(partials) / jnp.float32(B * O)


# speedup vs baseline: 1.0648x; 1.0385x over previous
import functools

import jax
import jax.numpy as jnp
from jax.experimental import pallas as pl
from jax.experimental.pallas import tpu as pltpu

_LANE = 128
_SUBLANE = 8


def _mse_partial_kernel(x_ref, y_ref, w1t_ref, b1_ref, w2_ref, b2_ref,
                        part_ref, *, tile_b, inv_n):
    y = y_ref[...]
    w2 = w2_ref[...]
    b2 = b2_ref[...]

    ht = jax.lax.dot_general(
        w1t_ref[...], x_ref[...], (((1,), (1,)), ((), ())),
        preferred_element_type=jnp.float32)
    ht = jax.nn.sigmoid(ht + b1_ref[...].T)

    s = jax.lax.dot_general(ht, ht, (((1,), (1,)), ((), ())),
                            preferred_element_type=jnp.float32)
    w2g = jax.lax.dot_general(w2, w2, (((1,), (1,)), ((), ())),
                              preferred_element_type=jnp.float32)
    sum_a2 = jnp.sum(s * w2g)

    g = jnp.dot(ht, y, preferred_element_type=jnp.float32)
    cross_ay = jnp.sum(g * w2)

    hsum = jnp.sum(ht, axis=1, keepdims=True)
    w2b2 = jnp.dot(w2, b2.T, preferred_element_type=jnp.float32)
    cross_ab2 = jnp.sum(hsum * w2b2)

    b2sq = jnp.sum(b2 * b2)
    y_term = jnp.sum(y * (y - 2.0 * b2))

    part = (sum_a2 + 2.0 * cross_ab2 + tile_b * b2sq
            - 2.0 * cross_ay + y_term)

    i = pl.program_id(0)

    @pl.when(i == 0)
    def _():
        part_ref[...] = jnp.zeros_like(part_ref)

    part_ref[...] = part_ref[...] + part

    @pl.when(i == pl.num_programs(0) - 1)
    def _():
        part_ref[...] = part_ref[...] * inv_n


def _pick_tile(b):
    for tb in (16384, 8192, 4096, 2048, 1024, 512, 256, 128, 64, 32, 16, 8):
        if b % tb == 0:
            return tb
    return b


def kernel(x, y, w1, b1, w2, b2):
    B, D = x.shape
    H = w1.shape[1]
    O = w2.shape[1]

    TB = _pick_tile(B)
    nb = B // TB

    def resident(shape):
        return pl.BlockSpec(shape, lambda i: (0, 0))

    block_bytes = (TB * D + TB * O) * 4
    vmem_limit = min(64 * 1024 * 1024, 2 * block_bytes + (8 << 20))

    loss = pl.pallas_call(
        functools.partial(_mse_partial_kernel, tile_b=float(TB),
                          inv_n=1.0 / float(B * O)),
        out_shape=jax.ShapeDtypeStruct((1, 1), jnp.float32),
        grid=(nb,),
        in_specs=[
            pl.BlockSpec((TB, D), lambda i: (i, 0)),
            pl.BlockSpec((TB, O), lambda i: (i, 0)),
            resident((H, D)),
            resident((1, H)),
            resident((H, O)),
            resident((1, O)),
        ],
        out_specs=pl.BlockSpec((1, 1), lambda i: (0, 0)),
        compiler_params=pltpu.CompilerParams(
            dimension_semantics=("arbitrary",),
            vmem_limit_bytes=vmem_limit,
        ),
    )(x, y, w1.T, b1, w2, b2)
    return jnp.reshape(loss, ())


# speedup vs baseline: 1.0773x; 1.0118x over previous
import functools

import jax
import jax.numpy as jnp
from jax.experimental import pallas as pl
from jax.experimental.pallas import tpu as pltpu

_LANE = 128
_SUBLANE = 8


def _mse_partial_kernel(x_ref, y_ref, w1t_ref, b1_ref, w2_ref, b2_ref,
                        part_ref, *, tile_b, inv_n):
    y = y_ref[...]
    w2 = w2_ref[...]
    b2 = b2_ref[...]

    ht = jax.lax.dot_general(
        w1t_ref[...], x_ref[...], (((1,), (1,)), ((), ())),
        preferred_element_type=jnp.float32)
    ht = jax.nn.sigmoid(ht + b1_ref[...].T)

    s = jax.lax.dot_general(ht, ht, (((1,), (1,)), ((), ())),
                            preferred_element_type=jnp.float32)
    w2g = jax.lax.dot_general(w2, w2, (((1,), (1,)), ((), ())),
                              preferred_element_type=jnp.float32)
    sum_a2 = jnp.sum(s * w2g)

    g = jnp.dot(ht, y, preferred_element_type=jnp.float32)
    cross_ay = jnp.sum(g * w2)

    hsum = jnp.sum(ht, axis=1, keepdims=True)
    w2b2 = jnp.dot(w2, b2.T, preferred_element_type=jnp.float32)
    cross_ab2 = jnp.sum(hsum * w2b2)

    b2sq = jnp.sum(b2 * b2)
    y_term = jnp.sum(y * (y - 2.0 * b2))

    part = (sum_a2 + 2.0 * cross_ab2 + tile_b * b2sq
            - 2.0 * cross_ay + y_term)

    i = pl.program_id(0)

    @pl.when(i == 0)
    def _():
        part_ref[...] = jnp.zeros_like(part_ref)

    part_ref[...] = part_ref[...] + part

    @pl.when(i == pl.num_programs(0) - 1)
    def _():
        part_ref[...] = part_ref[...] * inv_n


def _pick_tile(b):
    for tb in (8192, 4096, 2048, 1024, 512, 256, 128, 64, 32, 16, 8):
        if b % tb == 0:
            return tb
    return b


def kernel(x, y, w1, b1, w2, b2):
    B, D = x.shape
    H = w1.shape[1]
    O = w2.shape[1]

    TB = _pick_tile(B)
    nb = B // TB

    def resident(shape):
        return pl.BlockSpec(shape, lambda i: (0, 0))

    block_bytes = (TB * D + TB * O) * 4
    vmem_limit = min(64 * 1024 * 1024, 2 * block_bytes + (8 << 20))

    loss = pl.pallas_call(
        functools.partial(_mse_partial_kernel, tile_b=float(TB),
                          inv_n=1.0 / float(B * O)),
        out_shape=jax.ShapeDtypeStruct((1, 1), jnp.float32),
        grid=(nb,),
        in_specs=[
            pl.BlockSpec((TB, D), lambda i: (i, 0)),
            pl.BlockSpec((TB, O), lambda i: (i, 0)),
            resident((H, D)),
            resident((1, H)),
            resident((H, O)),
            resident((1, O)),
        ],
        out_specs=pl.BlockSpec((1, 1), lambda i: (0, 0)),
        compiler_params=pltpu.CompilerParams(
            dimension_semantics=("arbitrary",),
            vmem_limit_bytes=vmem_limit,
        ),
    )(x, y, w1.T, b1, w2, b2)
    return jnp.reshape(loss, ())
